# baseline (speedup 1.0000x reference)
import jax
import jax.numpy as jnp
from jax.experimental import pallas as pl
from jax.experimental.pallas import tpu as pltpu


def _copy_kernel(blocks_ref, out_ref):
    out_ref[...] = blocks_ref[...]


def kernel(features, blocks, cluster_centers, W1, b1, W2, b2, epoch, max_epochs):
    N, L = blocks.shape
    ROWS = 256
    return pl.pallas_call(
        _copy_kernel,
        grid=(N // ROWS,),
        in_specs=[pl.BlockSpec((ROWS, L), lambda i: (i, 0))],
        out_specs=pl.BlockSpec((ROWS, L), lambda i: (i, 0)),
        out_shape=jax.ShapeDtypeStruct((N, L), blocks.dtype),
        compiler_params=pltpu.CompilerParams(
            dimension_semantics=("arbitrary",),
        ),
    )(blocks)


# speedup vs baseline: 1.8190x; 1.8190x over previous
import jax
import jax.numpy as jnp
from jax.experimental import pallas as pl
from jax.experimental.pallas import tpu as pltpu


def _copy_kernel(blocks_ref, out_ref):
    out_ref[...] = blocks_ref[...]


def kernel(features, blocks, cluster_centers, W1, b1, W2, b2, epoch, max_epochs):
    N, L = blocks.shape
    ROWS = 1024
    return pl.pallas_call(
        _copy_kernel,
        grid=(N // ROWS,),
        in_specs=[pl.BlockSpec((ROWS, L), lambda i: (i, 0))],
        out_specs=pl.BlockSpec((ROWS, L), lambda i: (i, 0)),
        out_shape=jax.ShapeDtypeStruct((N, L), blocks.dtype),
        compiler_params=pltpu.CompilerParams(
            dimension_semantics=("arbitrary",),
        ),
    )(blocks)
